# Initial kernel scaffold; baseline (speedup 1.0000x reference)
import functools

import jax
import jax.numpy as jnp
from jax import lax
from jax.experimental import pallas as pl
from jax.experimental.pallas import tpu as pltpu
from jax.experimental.pallas import tpu_sc as plsc

N = 10000
E = 320000
D = 128

NC = 2
NS = 16
NW = NC * NS
CHUNK = 80
EPT = E // NW
NCHUNK = EPT // CHUNK
RPT = N // NS
BN_SCALE = float(1.0 / (1.0 + 1e-5) ** 0.5)

_sc_mesh = plsc.VectorSubcoreMesh(core_axis_name="c", subcore_axis_name="s")


@functools.partial(
    pl.kernel,
    out_type=jax.ShapeDtypeStruct((NC, N), jnp.float32),
    mesh=_sc_mesh,
    scratch_types=[
        pltpu.VMEM_SHARED((N,), jnp.float32),
        pltpu.VMEM((CHUNK,), jnp.int32),
        pltpu.VMEM((CHUNK,), jnp.float32),
        pltpu.VMEM((N,), jnp.float32),
    ],
)
def _deg_sc(dst_hbm, hist_hbm, hist_sp, dst_v, ones_v, stage_v):
    c = lax.axis_index("c")
    s = lax.axis_index("s")
    wid = c * NS + s

    def fill_ones(i, _):
        ones_v[pl.ds(i * 16, 16)] = jnp.ones((16,), jnp.float32)
        return 0

    lax.fori_loop(0, CHUNK // 16, fill_ones, 0)

    @pl.when(s == 0)
    def _():
        def zero(i, _):
            stage_v[pl.ds(i * 16, 16)] = jnp.zeros((16,), jnp.float32)
            return 0

        lax.fori_loop(0, N // 16, zero, 0)
        pltpu.sync_copy(stage_v, hist_sp)

    plsc.subcore_barrier()

    def body(i, _):
        base = wid * EPT + i * CHUNK
        pltpu.sync_copy(dst_hbm.at[pl.ds(base, CHUNK)], dst_v)
        pltpu.sync_copy(ones_v, hist_sp.at[dst_v], add=True)
        return 0

    lax.fori_loop(0, NCHUNK, body, 0)
    plsc.subcore_barrier()

    @pl.when(s == 0)
    def _():
        pltpu.sync_copy(hist_sp, stage_v)
        pltpu.sync_copy(stage_v, hist_hbm.at[c])


@functools.partial(
    pl.kernel,
    out_type=jax.ShapeDtypeStruct((NC, N, D), jnp.float32),
    mesh=_sc_mesh,
    scratch_types=[
        pltpu.VMEM_SHARED((N, D), jnp.float32),
        pltpu.VMEM((CHUNK,), jnp.int32),
        pltpu.VMEM((CHUNK,), jnp.int32),
        pltpu.VMEM((CHUNK, D), jnp.float32),
        pltpu.SemaphoreType.DMA,
    ],
)
def _agg_sc(h_hbm, src_hbm, dst_hbm, out_hbm, agg_sp, src_v, dst_v, rows_v, sem):
    c = lax.axis_index("c")
    s = lax.axis_index("s")
    wid = c * NS + s

    r0 = s * RPT
    pltpu.sync_copy(h_hbm.at[pl.ds(r0, RPT)], agg_sp.at[pl.ds(r0, RPT)])
    plsc.subcore_barrier()

    def body(i, _):
        base = wid * EPT + i * CHUNK
        pltpu.sync_copy(src_hbm.at[pl.ds(base, CHUNK)], src_v)
        pltpu.sync_copy(dst_hbm.at[pl.ds(base, CHUNK)], dst_v)
        pltpu.async_copy(h_hbm.at[src_v], rows_v, sem).wait()
        pltpu.sync_copy(rows_v, agg_sp.at[dst_v], add=True)
        return 0

    lax.fori_loop(0, NCHUNK, body, 0)
    plsc.subcore_barrier()
    pltpu.sync_copy(agg_sp.at[pl.ds(r0, RPT)], out_hbm.at[c, pl.ds(r0, RPT)])


def _dense1_body(x_ref, w_ref, hist_ref, h_ref, dinv_ref):
    deg = 1.0 + hist_ref[:, 0:1] + hist_ref[:, 1:2]
    dinv = lax.rsqrt(deg)
    h = jnp.dot(x_ref[...], w_ref[...], preferred_element_type=jnp.float32)
    h_ref[...] = h * dinv
    dinv_ref[...] = dinv


def _mid_body(agg_ref, h_ref, dinv_ref, b_ref, g_ref, be_ref, w_ref, out_ref):
    dinv = dinv_ref[...]
    t = dinv * (agg_ref[0] + agg_ref[1] - h_ref[...]) + b_ref[...]
    t = g_ref[...] * (t * BN_SCALE) + be_ref[...]
    t = jnp.maximum(t, 0.0)
    out_ref[...] = dinv * jnp.dot(t, w_ref[...],
                                  preferred_element_type=jnp.float32)


def _fin_body(agg_ref, h_ref, dinv_ref, b_ref, out_ref):
    out_ref[...] = (dinv_ref[...] * (agg_ref[0] + agg_ref[1] - h_ref[...])
                    + b_ref[...])


_dense1 = pl.pallas_call(
    _dense1_body,
    out_shape=(jax.ShapeDtypeStruct((N, D), jnp.float32),
               jax.ShapeDtypeStruct((N, 1), jnp.float32)),
)

_mid = pl.pallas_call(
    _mid_body,
    out_shape=jax.ShapeDtypeStruct((N, D), jnp.float32),
)

_fin = pl.pallas_call(
    _fin_body,
    out_shape=jax.ShapeDtypeStruct((N, D), jnp.float32),
)


def kernel(x, edge_index, W1, b1, g1, be1, W2, b2, g2, be2, W3, b3):
    src = edge_index[0]
    dst = edge_index[1]

    hist = _deg_sc(dst)
    histT = hist.T
    h1p, dinv = _dense1(x, W1, histT)
    agg1 = _agg_sc(h1p, src, dst)
    h2p = _mid(agg1, h1p, dinv, b1.reshape(1, D), g1.reshape(1, D),
               be1.reshape(1, D), W2)
    agg2 = _agg_sc(h2p, src, dst)
    h3p = _mid(agg2, h2p, dinv, b2.reshape(1, D), g2.reshape(1, D),
               be2.reshape(1, D), W3)
    agg3 = _agg_sc(h3p, src, dst)
    return _fin(agg3, h3p, dinv, b3.reshape(1, D))


# speedup vs baseline: 11.3521x; 11.3521x over previous
import functools

import jax
import jax.numpy as jnp
from jax import lax
from jax.experimental import pallas as pl
from jax.experimental.pallas import tpu as pltpu
from jax.experimental.pallas import tpu_sc as plsc

N = 10000
E = 320000
D = 128

NC = 2
NS = 16
NW = NC * NS
CHUNK = 80
EPT = E // NW
NCHUNK = EPT // CHUNK
RPT = 624
RREM = N - NS * RPT
BN_SCALE = float(1.0 / (1.0 + 1e-5) ** 0.5)

_sc_mesh = plsc.VectorSubcoreMesh(core_axis_name="c", subcore_axis_name="s")


@functools.partial(
    pl.kernel,
    out_type=jax.ShapeDtypeStruct((NC, N), jnp.float32),
    mesh=_sc_mesh,
    scratch_types=[
        pltpu.VMEM_SHARED((N,), jnp.float32),
        pltpu.VMEM((CHUNK,), jnp.int32),
        pltpu.VMEM((CHUNK,), jnp.float32),
        pltpu.VMEM((N,), jnp.float32),
    ],
)
def _deg_sc(dst_hbm, hist_hbm, hist_sp, dst_v, ones_v, stage_v):
    c = lax.axis_index("c")
    s = lax.axis_index("s")
    wid = c * NS + s

    def fill_ones(i, _):
        ones_v[pl.ds(i * 16, 16)] = jnp.ones((16,), jnp.float32)
        return 0

    lax.fori_loop(0, CHUNK // 16, fill_ones, 0)

    @pl.when(s == 0)
    def _():
        def zero(i, _):
            stage_v[pl.ds(i * 16, 16)] = jnp.zeros((16,), jnp.float32)
            return 0

        lax.fori_loop(0, N // 16, zero, 0)
        pltpu.sync_copy(stage_v, hist_sp)

    plsc.subcore_barrier()

    def body(i, _):
        base = wid * EPT + i * CHUNK
        pltpu.sync_copy(dst_hbm.at[pl.ds(base, CHUNK)], dst_v)
        pltpu.sync_copy(ones_v, hist_sp.at[dst_v], add=True)
        return 0

    lax.fori_loop(0, NCHUNK, body, 0)
    plsc.subcore_barrier()

    @pl.when(s == 0)
    def _():
        pltpu.sync_copy(hist_sp, stage_v)
        pltpu.sync_copy(stage_v, hist_hbm.at[c])


@functools.partial(
    pl.kernel,
    out_type=jax.ShapeDtypeStruct((NC, N, D), jnp.float32),
    mesh=_sc_mesh,
    scratch_types=[
        pltpu.VMEM_SHARED((N, D), jnp.float32),
        pltpu.VMEM((CHUNK,), jnp.int32),
        pltpu.VMEM((CHUNK,), jnp.int32),
        pltpu.VMEM((CHUNK, D), jnp.float32),
        pltpu.SemaphoreType.DMA,
    ],
)
def _agg_sc(h_hbm, src_hbm, dst_hbm, out_hbm, agg_sp, src_v, dst_v, rows_v, sem):
    c = lax.axis_index("c")
    s = lax.axis_index("s")
    wid = c * NS + s

    r0 = s * RPT
    pltpu.sync_copy(h_hbm.at[pl.ds(r0, RPT)], agg_sp.at[pl.ds(r0, RPT)])

    @pl.when(s == NS - 1)
    def _():
        pltpu.sync_copy(h_hbm.at[pl.ds(NS * RPT, RREM)],
                        agg_sp.at[pl.ds(NS * RPT, RREM)])

    plsc.subcore_barrier()

    def body(i, _):
        base = wid * EPT + i * CHUNK
        pltpu.sync_copy(src_hbm.at[pl.ds(base, CHUNK)], src_v)
        pltpu.sync_copy(dst_hbm.at[pl.ds(base, CHUNK)], dst_v)
        pltpu.async_copy(h_hbm.at[src_v], rows_v, sem).wait()
        pltpu.sync_copy(rows_v, agg_sp.at[dst_v], add=True)
        return 0

    lax.fori_loop(0, NCHUNK, body, 0)
    plsc.subcore_barrier()
    pltpu.sync_copy(agg_sp.at[pl.ds(r0, RPT)], out_hbm.at[c, pl.ds(r0, RPT)])

    @pl.when(s == NS - 1)
    def _():
        pltpu.sync_copy(agg_sp.at[pl.ds(NS * RPT, RREM)],
                        out_hbm.at[c, pl.ds(NS * RPT, RREM)])


def _dense1_body(x_ref, w_ref, hist_ref, h_ref, dinv_ref):
    deg = 1.0 + hist_ref[:, 0:1] + hist_ref[:, 1:2]
    dinv = lax.rsqrt(deg)
    h = jnp.dot(x_ref[...], w_ref[...], preferred_element_type=jnp.float32)
    h_ref[...] = h * dinv
    dinv_ref[...] = dinv


def _mid_body(agg_ref, h_ref, dinv_ref, b_ref, g_ref, be_ref, w_ref, out_ref):
    dinv = dinv_ref[...]
    t = dinv * (agg_ref[0] + agg_ref[1] - h_ref[...]) + b_ref[...]
    t = g_ref[...] * (t * BN_SCALE) + be_ref[...]
    t = jnp.maximum(t, 0.0)
    out_ref[...] = dinv * jnp.dot(t, w_ref[...],
                                  preferred_element_type=jnp.float32)


def _fin_body(agg_ref, h_ref, dinv_ref, b_ref, out_ref):
    out_ref[...] = (dinv_ref[...] * (agg_ref[0] + agg_ref[1] - h_ref[...])
                    + b_ref[...])


_dense1 = pl.pallas_call(
    _dense1_body,
    out_shape=(jax.ShapeDtypeStruct((N, D), jnp.float32),
               jax.ShapeDtypeStruct((N, 1), jnp.float32)),
)

_mid = pl.pallas_call(
    _mid_body,
    out_shape=jax.ShapeDtypeStruct((N, D), jnp.float32),
)

_fin = pl.pallas_call(
    _fin_body,
    out_shape=jax.ShapeDtypeStruct((N, D), jnp.float32),
)


def kernel(x, edge_index, W1, b1, g1, be1, W2, b2, g2, be2, W3, b3):
    src = edge_index[0]
    dst = edge_index[1]

    hist = _deg_sc(dst)
    histT = hist.T
    h1p, dinv = _dense1(x, W1, histT)
    agg1 = _agg_sc(h1p, src, dst)
    h2p = _mid(agg1, h1p, dinv, b1.reshape(1, D), g1.reshape(1, D),
               be1.reshape(1, D), W2)
    agg2 = _agg_sc(h2p, src, dst)
    h3p = _mid(agg2, h2p, dinv, b2.reshape(1, D), g2.reshape(1, D),
               be2.reshape(1, D), W3)
    agg3 = _agg_sc(h3p, src, dst)
    return _fin(agg3, h3p, dinv, b3.reshape(1, D))
